# baseline (speedup 1.0000x reference)
import functools

import jax
import jax.numpy as jnp
from jax import lax
from jax.experimental import pallas as pl
from jax.experimental.pallas import tpu as pltpu
from jax.experimental.pallas import tpu_sc as plsc

_NC, _NS, _L = 2, 16, 16
_NT = _NC * _NS
_B, _F, _D = 4096, 26, 64
_RB = 100096
_HALF = _RB // 2
_NK = _F * 4


def _body(ids_hbm, offs_hbm, tab_hbm, out_hbm,
          offs_v, idc_v, rel0_v, rel1_v, valA, valB, bufA, bufB,
          semA, semB, semOA, semOB):
    c = lax.axis_index("c")
    s = lax.axis_index("s")
    t = c * _NS + s
    pltpu.sync_copy(offs_hbm, offs_v)
    bufA[pl.ds(_HALF, _L)] = lax.full((_L,), 0.0, jnp.float32)
    bufB[pl.ds(_HALF, _L)] = lax.full((_L,), 0.0, jnp.float32)

    def off_at(f):
        return offs_v[0, pl.ds(f, _L)][0]

    def chunk_src(k):
        f = lax.div(k, 4)
        r = lax.rem(k, 4)
        j = lax.div(r, 2)
        h = lax.rem(r, 2)
        off = off_at(f)
        rb = pl.multiple_of(lax.bitwise_and(off, ~127), 128)
        return tab_hbm.at[t + j * _NT, pl.ds(rb + h * _HALF, _HALF)]

    pltpu.async_copy(chunk_src(0), bufA.at[pl.ds(0, _HALF)], semA)
    pltpu.async_copy(chunk_src(1), bufB.at[pl.ds(0, _HALF)], semB)

    def field_step(f, carry):
        pltpu.sync_copy(ids_hbm.at[f], idc_v)
        off = off_at(f)
        base = off - lax.bitwise_and(off, ~127)
        sent = lax.full((_L,), _HALF, jnp.int32)
        lim = lax.full((_L,), _HALF, jnp.uint32)
        for g in range(_B // _L):
            sl = pl.ds(g * _L, _L)
            rel = idc_v[0, sl] + base
            in0 = lax.lt(plsc.bitcast(rel, jnp.uint32), lim)
            rel0_v[0, sl] = lax.select(in0, rel, sent)
            relm = rel - _HALF
            in1 = lax.lt(plsc.bitcast(relm, jnp.uint32), lim)
            rel1_v[0, sl] = lax.select(in1, relm, sent)

        for j, val_v, semO in ((0, valA, semOA), (1, valB, semOB)):
            k = f * 4 + j * 2
            dst = out_hbm.at[f, t + j * _NT]

            @pl.when(f > 0)
            def _():
                pltpu.make_async_copy(val_v.at[0], dst, semO).wait()

            pltpu.make_async_copy(chunk_src(k),
                                  bufA.at[pl.ds(0, _HALF)], semA).wait()
            for g in range(_B // _L):
                sl = pl.ds(g * _L, _L)
                val_v[0, sl] = plsc.load_gather(bufA, [rel0_v[0, sl]])

            @pl.when(k + 2 < _NK)
            def _():
                pltpu.async_copy(chunk_src(k + 2),
                                 bufA.at[pl.ds(0, _HALF)], semA)

            pltpu.make_async_copy(chunk_src(k + 1),
                                  bufB.at[pl.ds(0, _HALF)], semB).wait()
            for g in range(_B // _L):
                sl = pl.ds(g * _L, _L)
                val_v[0, sl] = val_v[0, sl] + plsc.load_gather(
                    bufB, [rel1_v[0, sl]])

            @pl.when(k + 3 < _NK)
            def _():
                pltpu.async_copy(chunk_src(k + 3),
                                 bufB.at[pl.ds(0, _HALF)], semB)

            pltpu.async_copy(val_v.at[0], dst, semO)
        return carry

    lax.fori_loop(0, _F, field_step, 0)
    pltpu.make_async_copy(valA.at[0], out_hbm.at[_F - 1, t], semOA).wait()
    pltpu.make_async_copy(valB.at[0], out_hbm.at[_F - 1, t + _NT],
                          semOB).wait()


@jax.jit
def _sc_gather(ids_t, offs, tab_t):
    mesh = plsc.VectorSubcoreMesh(core_axis_name="c", subcore_axis_name="s")
    f = pl.kernel(
        _body,
        out_type=jax.ShapeDtypeStruct((_F, _D, _B), jnp.float32),
        mesh=mesh,
        scratch_types=[
            pltpu.VMEM((1, 48), jnp.int32),
            pltpu.VMEM((1, _B), jnp.int32),
            pltpu.VMEM((1, _B), jnp.int32),
            pltpu.VMEM((1, _B), jnp.int32),
            pltpu.VMEM((1, _B), jnp.float32),
            pltpu.VMEM((1, _B), jnp.float32),
            pltpu.VMEM((_HALF + _L,), jnp.float32),
            pltpu.VMEM((_HALF + _L,), jnp.float32),
            pltpu.SemaphoreType.DMA,
            pltpu.SemaphoreType.DMA,
            pltpu.SemaphoreType.DMA,
            pltpu.SemaphoreType.DMA,
        ],
        compiler_params=pltpu.CompilerParams(needs_layout_passes=False),
    )
    return f(ids_t, offs, tab_t)


def kernel(hash_ids, table, offsets):
    ids_t = hash_ids.astype(jnp.int32).T.reshape(_F, 1, _B)
    offs = jnp.zeros((1, 48), jnp.int32).at[0, :_F].set(
        offsets.astype(jnp.int32))
    out = _sc_gather(ids_t, offs, table.T)
    return out.transpose(2, 0, 1)
